# Initial kernel scaffold; baseline (speedup 1.0000x reference)
import jax
import jax.numpy as jnp
from jax.experimental import pallas as pl


def kernel(preds, target):
    raise NotImplementedError("write your pallas kernel here")


# speedup vs baseline: 20.1557x; 20.1557x over previous
import functools

import jax
import jax.numpy as jnp
from jax import lax
from jax.experimental import pallas as pl
from jax.experimental.pallas import tpu as pltpu
from jax.experimental.pallas import tpu_sc as plsc

N = 4_000_000
K = 100
NC = 2
NS = 16
NW = NC * NS
PER_W = N // NW
FULL_CH = 16384
N_FULL = PER_W // FULL_CH
TAIL_CH = PER_W - N_FULL * FULL_CH
TAIL_VECS = TAIL_CH // 16
TAIL_REM = TAIL_CH - TAIL_VECS * 16
HBITS = 10
HBINS = 1 << HBITS
CAP = 512

_mesh = plsc.VectorSubcoreMesh(core_axis_name="c", subcore_axis_name="s")
_params = pltpu.CompilerParams(needs_layout_passes=False)


def _wid():
    return lax.axis_index("s") * NC + lax.axis_index("c")


def _key16(p):
    b = lax.bitcast_convert_type(p, jnp.uint32)
    top = b >> jnp.uint32(31)
    flip = (jnp.uint32(0) - top) | jnp.uint32(0x80000000)
    return b ^ flip


def _memset_i32(ref, nvecs, value=0):
    zz = jnp.full((16,), value, dtype=jnp.int32)

    def body(i, carry):
        ref[pl.ds(i * 16, 16)] = zz
        return carry

    lax.fori_loop(0, nvecs, body, 0)


@functools.partial(
    pl.kernel,
    out_type=(
        jax.ShapeDtypeStruct((NW, HBINS), jnp.int32),
        jax.ShapeDtypeStruct((NW, 16), jnp.int32),
    ),
    mesh=_mesh,
    compiler_params=_params,
    scratch_types=[
        pltpu.VMEM((FULL_CH,), jnp.float32),
        pltpu.VMEM((FULL_CH,), jnp.int32),
        pltpu.VMEM((HBINS * 16,), jnp.int32),
        pltpu.VMEM((HBINS,), jnp.int32),
        pltpu.VMEM((16,), jnp.int32),
    ],
)
def _scan1(preds_hbm, target_hbm, hist_out, pos_out, pbuf, tbuf, hist, hred, stage):
    w = _wid()
    base = w * PER_W
    lanes = lax.iota(jnp.int32, 16)
    ones = jnp.full((16,), 1, dtype=jnp.int32)

    _memset_i32(hist, HBINS)

    def do_vec(p, t, poscnt, valid=None):
        key = _key16(p)
        bucket = (key >> jnp.uint32(32 - HBITS)).astype(jnp.int32)
        idx = bucket * 16 + lanes
        negm = t == 0
        posm = t == 1
        if valid is not None:
            negm = negm & valid
            posm = posm & valid
        plsc.addupdate_scatter(hist, [idx], ones, mask=negm)
        return poscnt + jnp.where(posm, 1, 0)

    poscnt = jnp.zeros((16,), dtype=jnp.int32)
    for c in range(N_FULL + 1):
        ch = FULL_CH if c < N_FULL else TAIL_CH
        nv = ch // 16
        off = base + c * FULL_CH
        pltpu.sync_copy(preds_hbm.at[pl.ds(off, ch)], pbuf.at[pl.ds(0, ch)])
        pltpu.sync_copy(target_hbm.at[pl.ds(off, ch)], tbuf.at[pl.ds(0, ch)])

        def body(i, poscnt):
            p = pbuf[pl.ds(i * 16, 16)]
            t = tbuf[pl.ds(i * 16, 16)]
            return do_vec(p, t, poscnt)

        poscnt = lax.fori_loop(0, nv, body, poscnt)
        if c == N_FULL and TAIL_REM:
            p = pbuf[pl.ds(TAIL_VECS * 16, 16)]
            t = tbuf[pl.ds(TAIL_VECS * 16, 16)]
            poscnt = do_vec(p, t, poscnt, valid=lanes < TAIL_REM)

    def red(v, carry):
        acc = jnp.zeros((16,), dtype=jnp.int32)
        ibase = v * 256 + lanes * 16
        for l in range(16):
            acc = acc + plsc.load_gather(hist, [ibase + l])
        hred[pl.ds(v * 16, 16)] = acc
        return carry

    lax.fori_loop(0, HBINS // 16, red, 0)
    stage[pl.ds(0, 16)] = poscnt
    pltpu.sync_copy(hred, hist_out.at[w])
    pltpu.sync_copy(stage, pos_out.at[w])


@functools.partial(
    pl.kernel,
    out_type=(
        jax.ShapeDtypeStruct((NW, CAP), jnp.uint32),
        jax.ShapeDtypeStruct((NW, CAP), jnp.uint32),
        jax.ShapeDtypeStruct((NW, 16), jnp.int32),
    ),
    mesh=_mesh,
    compiler_params=_params,
    scratch_types=[
        pltpu.VMEM((FULL_CH,), jnp.float32),
        pltpu.VMEM((FULL_CH,), jnp.int32),
        pltpu.VMEM((NW, HBINS), jnp.int32),
        pltpu.VMEM((CAP,), jnp.uint32),
        pltpu.VMEM((CAP,), jnp.uint32),
        pltpu.VMEM((16,), jnp.int32),
    ],
)
def _scan2(preds_hbm, target_hbm, hist_hbm, negk_out, posk_out, cnt_out,
           pbuf, tbuf, hall, negbuf, posbuf, stage):
    w = _wid()
    base = w * PER_W
    lanes = lax.iota(jnp.int32, 16)

    pltpu.sync_copy(hist_hbm, hall)

    def merge_v(v, carry):
        popvec, cnt_above = carry
        vv = HBINS // 16 - 1 - v
        acc = jnp.zeros((16,), dtype=jnp.int32)
        for r in range(NW):
            acc = acc + hall[r, pl.ds(vv * 16, 16)]
        suf = lax.rev(lax.cumsum(lax.rev(acc, (0,)), axis=0), (0,))
        cnt_ge = suf + jnp.full((16,), cnt_above, dtype=jnp.int32)
        popvec = popvec + jnp.where(cnt_ge >= K, 1, 0)
        cnt_above = cnt_above + lax.reduce_sum(acc, axes=(0,))
        return popvec, cnt_above

    popvec, _ = lax.fori_loop(
        0, HBINS // 16, merge_v,
        (jnp.zeros((16,), dtype=jnp.int32), jnp.int32(0)))
    nbins_ge = lax.reduce_sum(popvec, axes=(0,))
    b1 = nbins_ge - 1
    lo1 = b1.astype(jnp.uint32) << jnp.uint32(32 - HBITS)
    lo1v = jnp.full((16,), lo1, dtype=jnp.uint32)

    _memset_i32(negbuf, CAP // 16)
    _memset_i32(posbuf, CAP // 16)

    def do_vec(p, t, carry, valid=None):
        noff, poff = carry
        key = _key16(p)
        ge = key >= lo1v
        negm = (t == 0) & ge
        posm = (t == 1) & ge
        if valid is not None:
            negm = negm & valid
            posm = posm & valid
        plsc.store_compressed(negbuf.at[pl.ds(noff, 16)], key, mask=negm)
        plsc.store_compressed(posbuf.at[pl.ds(poff, 16)], key, mask=posm)
        nadd = lax.reduce_sum(jnp.where(negm, 1, 0), axes=(0,))
        padd = lax.reduce_sum(jnp.where(posm, 1, 0), axes=(0,))
        noff = jnp.minimum(noff + nadd, CAP - 16)
        poff = jnp.minimum(poff + padd, CAP - 16)
        return noff, poff

    carry = (jnp.int32(0), jnp.int32(0))
    for c in range(N_FULL + 1):
        ch = FULL_CH if c < N_FULL else TAIL_CH
        nv = ch // 16
        off = base + c * FULL_CH
        pltpu.sync_copy(preds_hbm.at[pl.ds(off, ch)], pbuf.at[pl.ds(0, ch)])
        pltpu.sync_copy(target_hbm.at[pl.ds(off, ch)], tbuf.at[pl.ds(0, ch)])

        def body(i, carry):
            p = pbuf[pl.ds(i * 16, 16)]
            t = tbuf[pl.ds(i * 16, 16)]
            return do_vec(p, t, carry)

        carry = lax.fori_loop(0, nv, body, carry)
        if c == N_FULL and TAIL_REM:
            p = pbuf[pl.ds(TAIL_VECS * 16, 16)]
            t = tbuf[pl.ds(TAIL_VECS * 16, 16)]
            carry = do_vec(p, t, carry, valid=lanes < TAIL_REM)

    noff, poff = carry
    stage[pl.ds(0, 16)] = jnp.where(lanes == 0, noff, 0) + jnp.where(
        lanes == 1, poff, 0)
    pltpu.sync_copy(negbuf, negk_out.at[w])
    pltpu.sync_copy(posbuf, posk_out.at[w])
    pltpu.sync_copy(stage, cnt_out.at[w])


@functools.partial(
    pl.kernel,
    out_type=jax.ShapeDtypeStruct((16,), jnp.float32),
    mesh=_mesh,
    compiler_params=_params,
    scratch_types=[
        pltpu.VMEM((NW, CAP), jnp.uint32),
        pltpu.VMEM((NW, CAP), jnp.uint32),
        pltpu.VMEM((NW, 16), jnp.int32),
        pltpu.VMEM((NW, 16), jnp.int32),
        pltpu.VMEM((16,), jnp.float32),
    ],
)
def _select(negk_hbm, posk_hbm, cnt_hbm, pos_hbm, out_hbm,
            negv, posv, cntv, posc, stage):
    w = _wid()

    @pl.when(w == 0)
    def _():
        pltpu.sync_copy(negk_hbm, negv)
        pltpu.sync_copy(posk_hbm, posv)
        pltpu.sync_copy(cnt_hbm, cntv)
        pltpu.sync_copy(pos_hbm, posc)

        lanes = lax.iota(jnp.int32, 16)

        def acc_body(r, carry):
            pc, mx = carry
            pc = pc + posc[r, pl.ds(0, 16)]
            mx = jnp.maximum(mx, cntv[r, pl.ds(0, 16)])
            return pc, mx

        pcvec, mxvec = lax.fori_loop(
            0, NW, acc_body,
            (jnp.zeros((16,), jnp.int32), jnp.zeros((16,), jnp.int32)))
        num_pos = lax.reduce_sum(pcvec, axes=(0,))
        maxnc = lax.reduce_sum(jnp.where(lanes == 0, mxvec, 0), axes=(0,))
        maxpc = lax.reduce_sum(jnp.where(lanes == 1, mxvec, 0), axes=(0,))
        nvn = (maxnc + 15) >> 4
        nvp = (maxpc + 15) >> 4

        def count_gt(buf, thresh, nv):
            tv = jnp.full((16,), thresh, dtype=jnp.uint32)

            def row(r, cnt):
                def col(v, cnt):
                    x = buf[r, pl.ds(v * 16, 16)]
                    return cnt + jnp.where(x > tv, 1, 0)

                return lax.fori_loop(0, nv, col, cnt)

            cvec = lax.fori_loop(0, NW, row, jnp.zeros((16,), jnp.int32))
            return lax.reduce_sum(cvec, axes=(0,))

        def bis(_, carry):
            lo, hi = carry
            mid = lo + ((hi - lo) >> jnp.uint32(1))
            c = count_gt(negv, mid, nvn)
            take_hi = c < K
            lo = jnp.where(take_hi, lo, mid + jnp.uint32(1))
            hi = jnp.where(take_hi, mid, hi)
            return lo, hi

        kth, _ = lax.fori_loop(
            0, 32, bis, (jnp.uint32(0), jnp.uint32(0xFFFFFFFF)))

        hits = count_gt(posv, kth, nvp)
        hits_v = jnp.full((16,), hits, dtype=jnp.int32).astype(jnp.float32)
        npos_v = jnp.full((16,), num_pos, dtype=jnp.int32).astype(jnp.float32)
        stage[pl.ds(0, 16)] = hits_v / npos_v
        pltpu.sync_copy(stage, out_hbm)


def kernel(preds, target):
    hist, poscnt = _scan1(preds, target)
    negk, posk, cnts = _scan2(preds, target, hist)
    out = _select(negk, posk, cnts, poscnt)
    return out[0]


# speedup vs baseline: 45.5318x; 2.2590x over previous
import functools

import jax
import jax.numpy as jnp
from jax import lax
from jax.experimental import pallas as pl
from jax.experimental.pallas import tpu as pltpu
from jax.experimental.pallas import tpu_sc as plsc

N = 4_000_000
K = 100
NC = 2
NS = 16
NW = NC * NS
PER_W = N // NW
FULL_CH = 16384
N_FULL = PER_W // FULL_CH
TAIL_CH = PER_W - N_FULL * FULL_CH
UNROLL = 8
BLK = UNROLL * 16
TAIL_BLKS = TAIL_CH // BLK
TAIL_VECS = (TAIL_CH - TAIL_BLKS * BLK) // 16
TAIL_REM = TAIL_CH - TAIL_BLKS * BLK - TAIL_VECS * 16
HBITS = 10
HBINS = 1 << HBITS
LOW_BITS = 32 - HBITS
CAP = 512

_mesh = plsc.VectorSubcoreMesh(core_axis_name="c", subcore_axis_name="s")
_params = pltpu.CompilerParams(needs_layout_passes=False)


def _wid():
    return lax.axis_index("s") * NC + lax.axis_index("c")


def _key16(p):
    b = lax.bitcast_convert_type(p, jnp.uint32)
    top = b >> jnp.uint32(31)
    flip = (jnp.uint32(0) - top) | jnp.uint32(0x80000000)
    return b ^ flip


def _memset_i32(ref, nvecs, value=0):
    zz = jnp.full((16,), value, dtype=jnp.int32)

    def body(i, carry):
        ref[pl.ds(i * 16, 16)] = zz
        return carry

    lax.fori_loop(0, nvecs, body, 0)


def _chunk_loop(preds_hbm, target_hbm, base, pbufs, tbufs, sems, per_chunk):
    def issue(c):
        ch = FULL_CH if c < N_FULL else TAIL_CH
        off = base + c * FULL_CH
        b = c % 2
        dp = pltpu.async_copy(
            preds_hbm.at[pl.ds(off, ch)], pbufs[b].at[pl.ds(0, ch)], sems[b])
        dt = pltpu.async_copy(
            target_hbm.at[pl.ds(off, ch)], tbufs[b].at[pl.ds(0, ch)], sems[b])
        return dp, dt

    descs = [None, None]
    descs[0] = issue(0)
    for c in range(N_FULL + 1):
        if c + 1 <= N_FULL:
            descs[(c + 1) % 2] = issue(c + 1)
        dp, dt = descs[c % 2]
        dp.wait()
        dt.wait()
        nblk = (FULL_CH // BLK) if c < N_FULL else TAIL_BLKS
        per_chunk(c % 2, nblk, is_tail=(c == N_FULL))


@functools.partial(
    pl.kernel,
    out_type=jax.ShapeDtypeStruct((NW, HBINS), jnp.int32),
    mesh=_mesh,
    compiler_params=_params,
    scratch_types=[
        pltpu.VMEM((FULL_CH,), jnp.float32),
        pltpu.VMEM((FULL_CH,), jnp.float32),
        pltpu.VMEM((FULL_CH,), jnp.int32),
        pltpu.VMEM((FULL_CH,), jnp.int32),
        pltpu.VMEM((HBINS * 16,), jnp.int32),
        pltpu.VMEM((HBINS,), jnp.int32),
        pltpu.SemaphoreType.DMA,
        pltpu.SemaphoreType.DMA,
    ],
)
def _scan1(preds_hbm, target_hbm, hist_out, pbuf0, pbuf1, tbuf0, tbuf1, hist,
           hred, sem0, sem1):
    w = _wid()
    base = w * PER_W
    pbufs, tbufs = (pbuf0, pbuf1), (tbuf0, tbuf1)
    lanes = lax.iota(jnp.int32, 16)
    ones = jnp.full((16,), 1, dtype=jnp.int32)

    _memset_i32(hist, HBINS)

    def calc_vec(b, e16, valid=None):
        p = pbufs[b][pl.ds(e16, 16)]
        t = tbufs[b][pl.ds(e16, 16)]
        key = _key16(p)
        bucket = (key >> jnp.uint32(LOW_BITS)).astype(jnp.int32)
        idx = bucket * 16 + lanes
        negm = t == 0
        if valid is not None:
            negm = negm & valid
        return idx, negm

    def do_vec(b, e16, valid=None):
        idx, negm = calc_vec(b, e16, valid)
        plsc.addupdate_scatter(hist, [idx], ones, mask=negm)

    def per_chunk(b, nblk, is_tail):
        def blk(i, carry):
            pend = [calc_vec(b, i * BLK + u * 16) for u in range(UNROLL)]
            for idx, negm in pend:
                plsc.addupdate_scatter(hist, [idx], ones, mask=negm)
            return carry

        lax.fori_loop(0, nblk, blk, 0)
        if is_tail:
            for u in range(TAIL_VECS):
                do_vec(b, TAIL_BLKS * BLK + u * 16)
            do_vec(b, TAIL_BLKS * BLK + TAIL_VECS * 16,
                   valid=lanes < TAIL_REM)

    _chunk_loop(preds_hbm, target_hbm, base, pbufs, tbufs,
                (sem0, sem1), per_chunk)

    def red(v, carry):
        acc = jnp.zeros((16,), dtype=jnp.int32)
        ibase = v * 256 + lanes * 16
        for l in range(16):
            acc = acc + plsc.load_gather(hist, [ibase + l])
        hred[pl.ds(v * 16, 16)] = acc
        return carry

    lax.fori_loop(0, HBINS // 16, red, 0)
    pltpu.sync_copy(hred, hist_out.at[w])


@functools.partial(
    pl.kernel,
    out_type=(
        jax.ShapeDtypeStruct((NW, CAP), jnp.uint32),
        jax.ShapeDtypeStruct((NW, CAP), jnp.uint32),
        jax.ShapeDtypeStruct((NW, 16), jnp.int32),
    ),
    mesh=_mesh,
    compiler_params=_params,
    scratch_types=[
        pltpu.VMEM((FULL_CH,), jnp.float32),
        pltpu.VMEM((FULL_CH,), jnp.float32),
        pltpu.VMEM((FULL_CH,), jnp.int32),
        pltpu.VMEM((FULL_CH,), jnp.int32),
        pltpu.VMEM((NW, HBINS), jnp.int32),
        pltpu.VMEM((CAP,), jnp.uint32),
        pltpu.VMEM((CAP,), jnp.uint32),
        pltpu.VMEM((16,), jnp.int32),
        pltpu.SMEM((8,), jnp.int32),
        pltpu.SemaphoreType.DMA,
        pltpu.SemaphoreType.DMA,
    ],
)
def _scan2(preds_hbm, target_hbm, hist_hbm, negk_out, posk_out, cnt_out,
           pbuf0, pbuf1, tbuf0, tbuf1, hall, negbuf, posbuf, stage, offs,
           sem0, sem1):
    w = _wid()
    base = w * PER_W
    pbufs, tbufs = (pbuf0, pbuf1), (tbuf0, tbuf1)
    lanes = lax.iota(jnp.int32, 16)

    pltpu.sync_copy(hist_hbm, hall)

    def merge_v(v, carry):
        popvec, cnt_above = carry
        vv = HBINS // 16 - 1 - v
        acc = jnp.zeros((16,), dtype=jnp.int32)
        for r in range(NW):
            acc = acc + hall[r, pl.ds(vv * 16, 16)]
        suf = lax.rev(lax.cumsum(lax.rev(acc, (0,)), axis=0), (0,))
        cnt_ge = suf + jnp.full((16,), cnt_above, dtype=jnp.int32)
        popvec = popvec + jnp.where(cnt_ge >= K, 1, 0)
        cnt_above = cnt_above + lax.reduce_sum(acc, axes=(0,))
        return popvec, cnt_above

    popvec, total_neg = lax.fori_loop(
        0, HBINS // 16, merge_v,
        (jnp.zeros((16,), dtype=jnp.int32), jnp.int32(0)))
    nbins_ge = lax.reduce_sum(popvec, axes=(0,))
    b1 = nbins_ge - 1
    lo1 = b1.astype(jnp.uint32) << jnp.uint32(LOW_BITS)
    lo1v = jnp.full((16,), lo1, dtype=jnp.uint32)
    num_pos = N - total_neg

    _memset_i32(negbuf, CAP // 16)
    _memset_i32(posbuf, CAP // 16)
    offs[0] = jnp.int32(0)
    offs[1] = jnp.int32(0)

    def collect_vec(b, e16, key, valid=None):
        t = tbufs[b][pl.ds(e16, 16)]
        ge = key >= lo1v
        negm = (t == 0) & ge
        posm = (t == 1) & ge
        if valid is not None:
            negm = negm & valid
            posm = posm & valid
        noff = offs[0]
        poff = offs[1]
        plsc.store_compressed(negbuf.at[pl.ds(noff, 16)], key, mask=negm)
        plsc.store_compressed(posbuf.at[pl.ds(poff, 16)], key, mask=posm)
        nadd = lax.reduce_sum(jnp.where(negm, 1, 0), axes=(0,))
        padd = lax.reduce_sum(jnp.where(posm, 1, 0), axes=(0,))
        offs[0] = jnp.minimum(noff + nadd, CAP - 16)
        offs[1] = jnp.minimum(poff + padd, CAP - 16)

    def per_chunk(b, nblk, is_tail):
        def blk(i, carry):
            keys = []
            mx = None
            for u in range(UNROLL):
                p = pbufs[b][pl.ds(i * BLK + u * 16, 16)]
                key = _key16(p)
                keys.append(key)
                mx = key if mx is None else jnp.maximum(mx, key)
            anyhit = lax.reduce_max(mx, axes=(0,)) >= lo1

            @pl.when(anyhit)
            def _():
                for u in range(UNROLL):
                    collect_vec(b, i * BLK + u * 16, keys[u])

            return carry

        lax.fori_loop(0, nblk, blk, 0)
        if is_tail:
            for u in range(TAIL_VECS):
                e16 = TAIL_BLKS * BLK + u * 16
                collect_vec(b, e16, _key16(pbufs[b][pl.ds(e16, 16)]))
            e16 = TAIL_BLKS * BLK + TAIL_VECS * 16
            collect_vec(b, e16, _key16(pbufs[b][pl.ds(e16, 16)]),
                        valid=lanes < TAIL_REM)

    _chunk_loop(preds_hbm, target_hbm, base, pbufs, tbufs,
                (sem0, sem1), per_chunk)

    lo1_i32 = lax.bitcast_convert_type(lo1v, jnp.int32)
    meta = jnp.where(lanes == 0, offs[0], 0)
    meta = meta + jnp.where(lanes == 1, offs[1], 0)
    meta = meta + jnp.where(lanes == 2, lo1_i32, 0)
    meta = meta + jnp.where(lanes == 3, num_pos, 0)
    stage[pl.ds(0, 16)] = meta
    pltpu.sync_copy(negbuf, negk_out.at[w])
    pltpu.sync_copy(posbuf, posk_out.at[w])
    pltpu.sync_copy(stage, cnt_out.at[w])


@functools.partial(
    pl.kernel,
    out_type=jax.ShapeDtypeStruct((16,), jnp.float32),
    mesh=_mesh,
    compiler_params=_params,
    scratch_types=[
        pltpu.VMEM((NW, CAP), jnp.uint32),
        pltpu.VMEM((NW, CAP), jnp.uint32),
        pltpu.VMEM((NW, 16), jnp.int32),
        pltpu.VMEM((16,), jnp.float32),
    ],
)
def _select(negk_hbm, posk_hbm, cnt_hbm, out_hbm, negv, posv, cntv, stage):
    w = _wid()

    @pl.when(w == 0)
    def _():
        pltpu.sync_copy(negk_hbm, negv)
        pltpu.sync_copy(posk_hbm, posv)
        pltpu.sync_copy(cnt_hbm, cntv)

        lanes = lax.iota(jnp.int32, 16)

        def acc_body(r, mx):
            return jnp.maximum(mx, cntv[r, pl.ds(0, 16)])

        mxvec = lax.fori_loop(0, NW, acc_body, jnp.zeros((16,), jnp.int32))
        row0 = cntv[0, pl.ds(0, 16)]
        maxnc = lax.reduce_sum(jnp.where(lanes == 0, mxvec, 0), axes=(0,))
        maxpc = lax.reduce_sum(jnp.where(lanes == 1, mxvec, 0), axes=(0,))
        lo1 = lax.reduce_sum(
            jnp.where(lanes == 2, row0, 0), axes=(0,)).astype(jnp.uint32)
        num_pos = lax.reduce_sum(jnp.where(lanes == 3, row0, 0), axes=(0,))
        nvn = (maxnc + 15) >> 4
        nvp = (maxpc + 15) >> 4

        def count_gt(buf, thresh, nv):
            tv = jnp.full((16,), thresh, dtype=jnp.uint32)

            def row(r, cnt):
                def col(v, cnt):
                    x = buf[r, pl.ds(v * 16, 16)]
                    return cnt + jnp.where(x > tv, 1, 0)

                return lax.fori_loop(0, nv, col, cnt)

            cvec = lax.fori_loop(0, NW, row, jnp.zeros((16,), jnp.int32))
            return lax.reduce_sum(cvec, axes=(0,))

        def bis(_, carry):
            lo, hi = carry
            mid = lo + ((hi - lo) >> jnp.uint32(1))
            c = count_gt(negv, mid, nvn)
            take_hi = c < K
            lo = jnp.where(take_hi, lo, mid + jnp.uint32(1))
            hi = jnp.where(take_hi, mid, hi)
            return lo, hi

        kth, _ = lax.fori_loop(
            0, LOW_BITS, bis,
            (lo1, lo1 + jnp.uint32((1 << LOW_BITS) - 1)))

        hits = count_gt(posv, kth, nvp)
        hits_v = jnp.full((16,), hits, dtype=jnp.int32).astype(jnp.float32)
        npos_v = jnp.full((16,), num_pos, dtype=jnp.int32).astype(jnp.float32)
        stage[pl.ds(0, 16)] = hits_v / npos_v
        pltpu.sync_copy(stage, out_hbm)


def kernel(preds, target):
    hist = _scan1(preds, target)
    negk, posk, cnts = _scan2(preds, target, hist)
    out = _select(negk, posk, cnts)
    return out[0]


# speedup vs baseline: 45.6144x; 1.0018x over previous
import functools

import jax
import jax.numpy as jnp
from jax import lax
from jax.experimental import pallas as pl
from jax.experimental.pallas import tpu as pltpu
from jax.experimental.pallas import tpu_sc as plsc

N = 4_000_000
K = 100
NC = 2
NS = 16
NW = NC * NS
PER_W = N // NW
FULL_CH = 16384
N_FULL = PER_W // FULL_CH
TAIL_CH = PER_W - N_FULL * FULL_CH
UNROLL = 8
BLK = UNROLL * 16
TAIL_BLKS = TAIL_CH // BLK
TAIL_VECS = (TAIL_CH - TAIL_BLKS * BLK) // 16
TAIL_REM = TAIL_CH - TAIL_BLKS * BLK - TAIL_VECS * 16
HBITS = 14
HBINS = 1 << HBITS
LOW_BITS = 32 - HBITS
SLICE = HBINS // NS
CAP = 256

_mesh = plsc.VectorSubcoreMesh(core_axis_name="c", subcore_axis_name="s")
_params = pltpu.CompilerParams(needs_layout_passes=False)


def _wid():
    return lax.axis_index("s") * NC + lax.axis_index("c")


def _key16(p):
    b = lax.bitcast_convert_type(p, jnp.uint32)
    top = b >> jnp.uint32(31)
    flip = (jnp.uint32(0) - top) | jnp.uint32(0x80000000)
    return b ^ flip


def _memset_i32(ref, nvecs, value=0):
    zz = jnp.full((16,), value, dtype=jnp.int32)

    def body(i, carry):
        ref[pl.ds(i * 16, 16)] = zz
        return carry

    lax.fori_loop(0, nvecs, body, 0)


def _chunk_loop(preds_hbm, target_hbm, base, pbufs, tbufs, sems, per_chunk):
    def issue(c):
        ch = FULL_CH if c < N_FULL else TAIL_CH
        off = base + c * FULL_CH
        b = c % 2
        dp = pltpu.async_copy(
            preds_hbm.at[pl.ds(off, ch)], pbufs[b].at[pl.ds(0, ch)], sems[b])
        dt = pltpu.async_copy(
            target_hbm.at[pl.ds(off, ch)], tbufs[b].at[pl.ds(0, ch)], sems[b])
        return dp, dt

    descs = [None, None]
    descs[0] = issue(0)
    for c in range(N_FULL + 1):
        if c + 1 <= N_FULL:
            descs[(c + 1) % 2] = issue(c + 1)
        dp, dt = descs[c % 2]
        dp.wait()
        dt.wait()
        nblk = (FULL_CH // BLK) if c < N_FULL else TAIL_BLKS
        per_chunk(c % 2, nblk, is_tail=(c == N_FULL))


@functools.partial(
    pl.kernel,
    out_type=jax.ShapeDtypeStruct((NC, HBINS), jnp.int32),
    mesh=_mesh,
    compiler_params=_params,
    scratch_types=[
        pltpu.VMEM((FULL_CH,), jnp.float32),
        pltpu.VMEM((FULL_CH,), jnp.float32),
        pltpu.VMEM((FULL_CH,), jnp.int32),
        pltpu.VMEM((FULL_CH,), jnp.int32),
        pltpu.VMEM((HBINS,), jnp.int32),
        pltpu.VMEM((NS, SLICE), jnp.int32),
        pltpu.VMEM((SLICE,), jnp.int32),
        pltpu.VMEM_SHARED((NS, HBINS), jnp.int32),
        pltpu.SemaphoreType.DMA,
        pltpu.SemaphoreType.DMA,
    ],
)
def _scan1(preds_hbm, target_hbm, hist_out, pbuf0, pbuf1, tbuf0, tbuf1, hist,
           colbuf, merged, slab, sem0, sem1):
    w = _wid()
    sid = lax.axis_index("s")
    cid = lax.axis_index("c")
    base = w * PER_W
    pbufs, tbufs = (pbuf0, pbuf1), (tbuf0, tbuf1)
    lanes = lax.iota(jnp.int32, 16)

    _memset_i32(hist, HBINS // 16)

    def calc_vec(b, e16, valid=None):
        p = pbufs[b][pl.ds(e16, 16)]
        t = tbufs[b][pl.ds(e16, 16)]
        key = _key16(p)
        bucket = (key >> jnp.uint32(LOW_BITS)).astype(jnp.int32)
        negm = t == 0
        if valid is not None:
            negm = negm & valid
        cnt, last = plsc.scan_count(bucket, mask=negm)
        return bucket, cnt, last

    def do_vec(b, e16, valid=None):
        bucket, cnt, last = calc_vec(b, e16, valid)
        plsc.addupdate_scatter(hist, [bucket], cnt, mask=last)

    def per_chunk(b, nblk, is_tail):
        def blk(i, carry):
            pend = [calc_vec(b, i * BLK + u * 16) for u in range(UNROLL)]
            for bucket, cnt, last in pend:
                plsc.addupdate_scatter(hist, [bucket], cnt, mask=last)
            return carry

        lax.fori_loop(0, nblk, blk, 0)
        if is_tail:
            for u in range(TAIL_VECS):
                do_vec(b, TAIL_BLKS * BLK + u * 16)
            do_vec(b, TAIL_BLKS * BLK + TAIL_VECS * 16,
                   valid=lanes < TAIL_REM)

    _chunk_loop(preds_hbm, target_hbm, base, pbufs, tbufs,
                (sem0, sem1), per_chunk)

    pltpu.sync_copy(hist, slab.at[sid])
    plsc.subcore_barrier()
    for r in range(NS):
        pltpu.sync_copy(slab.at[r, pl.ds(sid * SLICE, SLICE)], colbuf.at[r])

    def red(v, carry):
        acc = jnp.zeros((16,), dtype=jnp.int32)
        for r in range(NS):
            acc = acc + colbuf[r, pl.ds(v * 16, 16)]
        merged[pl.ds(v * 16, 16)] = acc
        return carry

    lax.fori_loop(0, SLICE // 16, red, 0)
    pltpu.sync_copy(merged, hist_out.at[cid, pl.ds(sid * SLICE, SLICE)])


@functools.partial(
    pl.kernel,
    out_type=(
        jax.ShapeDtypeStruct((NW, CAP), jnp.uint32),
        jax.ShapeDtypeStruct((NW, CAP), jnp.uint32),
        jax.ShapeDtypeStruct((NW, 16), jnp.int32),
    ),
    mesh=_mesh,
    compiler_params=_params,
    scratch_types=[
        pltpu.VMEM((FULL_CH,), jnp.float32),
        pltpu.VMEM((FULL_CH,), jnp.float32),
        pltpu.VMEM((FULL_CH,), jnp.int32),
        pltpu.VMEM((FULL_CH,), jnp.int32),
        pltpu.VMEM((NC, HBINS), jnp.int32),
        pltpu.VMEM((CAP,), jnp.uint32),
        pltpu.VMEM((CAP,), jnp.uint32),
        pltpu.VMEM((16,), jnp.int32),
        pltpu.SMEM((8,), jnp.int32),
        pltpu.SemaphoreType.DMA,
        pltpu.SemaphoreType.DMA,
    ],
)
def _scan2(preds_hbm, target_hbm, hist_hbm, negk_out, posk_out, cnt_out,
           pbuf0, pbuf1, tbuf0, tbuf1, hall, negbuf, posbuf, stage, offs,
           sem0, sem1):
    w = _wid()
    base = w * PER_W
    pbufs, tbufs = (pbuf0, pbuf1), (tbuf0, tbuf1)
    lanes = lax.iota(jnp.int32, 16)

    pltpu.sync_copy(hist_hbm, hall)

    def load_acc(vv):
        acc = hall[0, pl.ds(vv * 16, 16)]
        for r in range(1, NC):
            acc = acc + hall[r, pl.ds(vv * 16, 16)]
        return acc

    def walk_cond(carry):
        v, cnt_above = carry
        return (cnt_above < K) & (v < HBINS // 16)

    def walk_body(carry):
        v, cnt_above = carry
        acc = load_acc(HBINS // 16 - 1 - v)
        return v + 1, cnt_above + lax.reduce_sum(acc, axes=(0,))

    nv, cnt_incl = lax.while_loop(walk_cond, walk_body,
                                  (jnp.int32(0), jnp.int32(0)))
    vv_last = HBINS // 16 - nv
    acc = load_acc(vv_last)
    tot_last = lax.reduce_sum(acc, axes=(0,))
    suf = lax.rev(lax.cumsum(lax.rev(acc, (0,)), axis=0), (0,))
    cnt_ge = suf + jnp.full((16,), cnt_incl - tot_last, dtype=jnp.int32)
    m = lax.reduce_sum(jnp.where(cnt_ge >= K, 1, 0), axes=(0,))
    b1 = vv_last * 16 + m - 1

    def total_body(v, acc):
        return acc + load_acc(v)

    total_neg = lax.reduce_sum(
        lax.fori_loop(0, HBINS // 16, total_body,
                      jnp.zeros((16,), jnp.int32)), axes=(0,))
    lo1 = b1.astype(jnp.uint32) << jnp.uint32(LOW_BITS)
    lo1v = jnp.full((16,), lo1, dtype=jnp.uint32)
    num_pos = N - total_neg

    _memset_i32(negbuf, CAP // 16)
    _memset_i32(posbuf, CAP // 16)
    offs[0] = jnp.int32(0)
    offs[1] = jnp.int32(0)

    def collect_vec(b, e16, key, valid=None):
        t = tbufs[b][pl.ds(e16, 16)]
        ge = key >= lo1v
        negm = (t == 0) & ge
        posm = (t == 1) & ge
        if valid is not None:
            negm = negm & valid
            posm = posm & valid
        noff = offs[0]
        poff = offs[1]
        plsc.store_compressed(negbuf.at[pl.ds(noff, 16)], key, mask=negm)
        plsc.store_compressed(posbuf.at[pl.ds(poff, 16)], key, mask=posm)
        nadd = lax.reduce_sum(jnp.where(negm, 1, 0), axes=(0,))
        padd = lax.reduce_sum(jnp.where(posm, 1, 0), axes=(0,))
        offs[0] = jnp.minimum(noff + nadd, CAP - 16)
        offs[1] = jnp.minimum(poff + padd, CAP - 16)

    def per_chunk(b, nblk, is_tail):
        def blk(i, carry):
            keys = []
            mx = None
            for u in range(UNROLL):
                p = pbufs[b][pl.ds(i * BLK + u * 16, 16)]
                key = _key16(p)
                keys.append(key)
                mx = key if mx is None else jnp.maximum(mx, key)
            anyhit = lax.reduce_max(mx, axes=(0,)) >= lo1

            @pl.when(anyhit)
            def _():
                for u in range(UNROLL):
                    collect_vec(b, i * BLK + u * 16, keys[u])

            return carry

        lax.fori_loop(0, nblk, blk, 0)
        if is_tail:
            for u in range(TAIL_VECS):
                e16 = TAIL_BLKS * BLK + u * 16
                collect_vec(b, e16, _key16(pbufs[b][pl.ds(e16, 16)]))
            e16 = TAIL_BLKS * BLK + TAIL_VECS * 16
            collect_vec(b, e16, _key16(pbufs[b][pl.ds(e16, 16)]),
                        valid=lanes < TAIL_REM)

    _chunk_loop(preds_hbm, target_hbm, base, pbufs, tbufs,
                (sem0, sem1), per_chunk)

    lo1_i32 = lax.bitcast_convert_type(lo1v, jnp.int32)
    meta = jnp.where(lanes == 0, offs[0], 0)
    meta = meta + jnp.where(lanes == 1, offs[1], 0)
    meta = meta + jnp.where(lanes == 2, lo1_i32, 0)
    meta = meta + jnp.where(lanes == 3, num_pos, 0)
    stage[pl.ds(0, 16)] = meta
    pltpu.sync_copy(negbuf, negk_out.at[w])
    pltpu.sync_copy(posbuf, posk_out.at[w])
    pltpu.sync_copy(stage, cnt_out.at[w])


@functools.partial(
    pl.kernel,
    out_type=jax.ShapeDtypeStruct((16,), jnp.float32),
    mesh=_mesh,
    compiler_params=_params,
    scratch_types=[
        pltpu.VMEM((NW, CAP), jnp.uint32),
        pltpu.VMEM((NW, CAP), jnp.uint32),
        pltpu.VMEM((NW, 16), jnp.int32),
        pltpu.VMEM((16,), jnp.float32),
    ],
)
def _select(negk_hbm, posk_hbm, cnt_hbm, out_hbm, negv, posv, cntv, stage):
    w = _wid()

    @pl.when(w == 0)
    def _():
        pltpu.sync_copy(negk_hbm, negv)
        pltpu.sync_copy(posk_hbm, posv)
        pltpu.sync_copy(cnt_hbm, cntv)

        lanes = lax.iota(jnp.int32, 16)

        def acc_body(r, mx):
            return jnp.maximum(mx, cntv[r, pl.ds(0, 16)])

        mxvec = lax.fori_loop(0, NW, acc_body, jnp.zeros((16,), jnp.int32))
        row0 = cntv[0, pl.ds(0, 16)]
        maxnc = lax.reduce_sum(jnp.where(lanes == 0, mxvec, 0), axes=(0,))
        maxpc = lax.reduce_sum(jnp.where(lanes == 1, mxvec, 0), axes=(0,))
        lo1 = lax.reduce_sum(
            jnp.where(lanes == 2, row0, 0), axes=(0,)).astype(jnp.uint32)
        num_pos = lax.reduce_sum(jnp.where(lanes == 3, row0, 0), axes=(0,))
        nvn = (maxnc + 15) >> 4
        nvp = (maxpc + 15) >> 4

        def count_gt(buf, thresh, nv):
            tv = jnp.full((16,), thresh, dtype=jnp.uint32)

            def row(r, cnt):
                def col(v, cnt):
                    x = buf[r, pl.ds(v * 16, 16)]
                    return cnt + jnp.where(x > tv, 1, 0)

                return lax.fori_loop(0, nv, col, cnt)

            cvec = lax.fori_loop(0, NW, row, jnp.zeros((16,), jnp.int32))
            return lax.reduce_sum(cvec, axes=(0,))

        def bis(_, carry):
            lo, hi = carry
            mid = lo + ((hi - lo) >> jnp.uint32(1))
            c = count_gt(negv, mid, nvn)
            take_hi = c < K
            lo = jnp.where(take_hi, lo, mid + jnp.uint32(1))
            hi = jnp.where(take_hi, mid, hi)
            return lo, hi

        kth, _ = lax.fori_loop(
            0, LOW_BITS, bis,
            (lo1, lo1 + jnp.uint32((1 << LOW_BITS) - 1)))

        hits = count_gt(posv, kth, nvp)
        hits_v = jnp.full((16,), hits, dtype=jnp.int32).astype(jnp.float32)
        npos_v = jnp.full((16,), num_pos, dtype=jnp.int32).astype(jnp.float32)
        stage[pl.ds(0, 16)] = hits_v / npos_v
        pltpu.sync_copy(stage, out_hbm)


def kernel(preds, target):
    hist = _scan1(preds, target)
    negk, posk, cnts = _scan2(preds, target, hist)
    out = _select(negk, posk, cnts)
    return out[0]


# speedup vs baseline: 52.4390x; 1.1496x over previous
import functools

import jax
import jax.numpy as jnp
from jax import lax
from jax.experimental import pallas as pl
from jax.experimental.pallas import tpu as pltpu
from jax.experimental.pallas import tpu_sc as plsc

N = 4_000_000
K = 100
NC = 2
NS = 16
NW = NC * NS
PER_W = N // NW
FULL_CH = 16384
N_FULL = PER_W // FULL_CH
TAIL_CH = PER_W - N_FULL * FULL_CH
UNROLL = 8
BLK = UNROLL * 16
TAIL_BLKS = TAIL_CH // BLK
TAIL_VECS = (TAIL_CH - TAIL_BLKS * BLK) // 16
TAIL_REM = TAIL_CH - TAIL_BLKS * BLK - TAIL_VECS * 16
HBITS = 14
HBINS = 1 << HBITS
LOW_BITS = 32 - HBITS
SLICE = HBINS // NS
CAP = 256

_mesh = plsc.VectorSubcoreMesh(core_axis_name="c", subcore_axis_name="s")
_params = pltpu.CompilerParams(needs_layout_passes=False)


def _wid():
    return lax.axis_index("s") * NC + lax.axis_index("c")


def _key16(p):
    b = lax.bitcast_convert_type(p, jnp.uint32)
    top = b >> jnp.uint32(31)
    flip = (jnp.uint32(0) - top) | jnp.uint32(0x80000000)
    return b ^ flip


def _memset_i32(ref, nvecs, value=0):
    zz = jnp.full((16,), value, dtype=jnp.int32)

    def body(i, carry):
        ref[pl.ds(i * 16, 16)] = zz
        return carry

    lax.fori_loop(0, nvecs, body, 0)


def _chunk_loop(preds_hbm, target_hbm, base, pbufs, tbufs, sems, per_chunk):
    def issue(c):
        ch = FULL_CH if c < N_FULL else TAIL_CH
        off = base + c * FULL_CH
        b = c % 2
        dp = pltpu.async_copy(
            preds_hbm.at[pl.ds(off, ch)], pbufs[b].at[pl.ds(0, ch)], sems[b])
        dt = pltpu.async_copy(
            target_hbm.at[pl.ds(off, ch)], tbufs[b].at[pl.ds(0, ch)], sems[b])
        return dp, dt

    descs = [None, None]
    descs[0] = issue(0)
    for c in range(N_FULL + 1):
        if c + 1 <= N_FULL:
            descs[(c + 1) % 2] = issue(c + 1)
        dp, dt = descs[c % 2]
        dp.wait()
        dt.wait()
        nblk = (FULL_CH // BLK) if c < N_FULL else TAIL_BLKS
        per_chunk(c % 2, nblk, is_tail=(c == N_FULL))


@functools.partial(
    pl.kernel,
    out_type=(
        jax.ShapeDtypeStruct((NW, CAP), jnp.uint32),
        jax.ShapeDtypeStruct((NW, CAP), jnp.uint32),
        jax.ShapeDtypeStruct((NW, 16), jnp.int32),
    ),
    mesh=_mesh,
    compiler_params=_params,
    scratch_types=[
        pltpu.VMEM((FULL_CH,), jnp.float32),
        pltpu.VMEM((FULL_CH,), jnp.float32),
        pltpu.VMEM((FULL_CH,), jnp.int32),
        pltpu.VMEM((FULL_CH,), jnp.int32),
        pltpu.VMEM((HBINS,), jnp.int32),
        pltpu.VMEM((NS, SLICE), jnp.int32),
        pltpu.VMEM((CAP,), jnp.uint32),
        pltpu.VMEM((CAP,), jnp.uint32),
        pltpu.VMEM((16,), jnp.int32),
        pltpu.SMEM((8,), jnp.int32),
        pltpu.VMEM_SHARED((NS, HBINS), jnp.int32),
        pltpu.VMEM_SHARED((HBINS,), jnp.int32),
        pltpu.SemaphoreType.DMA,
        pltpu.SemaphoreType.DMA,
    ],
)
def _scan(preds_hbm, target_hbm, negk_out, posk_out, cnt_out,
          pbuf0, pbuf1, tbuf0, tbuf1, hist, colbuf, negbuf, posbuf, stage,
          offs, slab, merged_sh, sem0, sem1):
    w = _wid()
    sid = lax.axis_index("s")
    base = w * PER_W
    pbufs, tbufs = (pbuf0, pbuf1), (tbuf0, tbuf1)
    lanes = lax.iota(jnp.int32, 16)

    _memset_i32(hist, HBINS // 16)

    def calc_vec(b, e16, valid=None):
        p = pbufs[b][pl.ds(e16, 16)]
        t = tbufs[b][pl.ds(e16, 16)]
        key = _key16(p)
        bucket = (key >> jnp.uint32(LOW_BITS)).astype(jnp.int32)
        negm = t == 0
        if valid is not None:
            negm = negm & valid
        cnt, last = plsc.scan_count(bucket, mask=negm)
        return bucket, cnt, last

    def do_vec(b, e16, valid=None):
        bucket, cnt, last = calc_vec(b, e16, valid)
        plsc.addupdate_scatter(hist, [bucket], cnt, mask=last)

    def per_chunk1(b, nblk, is_tail):
        def blk(i, carry):
            pend = [calc_vec(b, i * BLK + u * 16) for u in range(UNROLL)]
            for bucket, cnt, last in pend:
                plsc.addupdate_scatter(hist, [bucket], cnt, mask=last)
            return carry

        lax.fori_loop(0, nblk, blk, 0)
        if is_tail:
            for u in range(TAIL_VECS):
                do_vec(b, TAIL_BLKS * BLK + u * 16)
            do_vec(b, TAIL_BLKS * BLK + TAIL_VECS * 16,
                   valid=lanes < TAIL_REM)

    _chunk_loop(preds_hbm, target_hbm, base, pbufs, tbufs,
                (sem0, sem1), per_chunk1)

    pltpu.sync_copy(hist, slab.at[sid])
    plsc.subcore_barrier()
    for r in range(NS):
        pltpu.sync_copy(slab.at[r, pl.ds(sid * SLICE, SLICE)], colbuf.at[r])

    def red(v, carry):
        acc = jnp.zeros((16,), dtype=jnp.int32)
        for r in range(NS):
            acc = acc + colbuf[r, pl.ds(v * 16, 16)]
        hist[pl.ds(sid * SLICE + v * 16, 16)] = acc
        return carry

    lax.fori_loop(0, SLICE // 16, red, 0)
    pltpu.sync_copy(hist.at[pl.ds(sid * SLICE, SLICE)],
                    merged_sh.at[pl.ds(sid * SLICE, SLICE)])
    plsc.subcore_barrier()
    pltpu.sync_copy(merged_sh, hist)

    def load_acc(vv):
        return hist[pl.ds(vv * 16, 16)]

    def walk_cond(carry):
        v, cnt_above = carry
        return (cnt_above < K) & (v < HBINS // 16)

    def walk_body(carry):
        v, cnt_above = carry
        acc = load_acc(HBINS // 16 - 1 - v)
        return v + 1, cnt_above + lax.reduce_sum(acc, axes=(0,))

    nv, cnt_incl = lax.while_loop(walk_cond, walk_body,
                                  (jnp.int32(0), jnp.int32(0)))
    vv_last = HBINS // 16 - nv
    acc = load_acc(vv_last)
    tot_last = lax.reduce_sum(acc, axes=(0,))
    suf = lax.rev(lax.cumsum(lax.rev(acc, (0,)), axis=0), (0,))
    cnt_ge = suf + jnp.full((16,), cnt_incl - tot_last, dtype=jnp.int32)
    m = lax.reduce_sum(jnp.where(cnt_ge >= K, 1, 0), axes=(0,))
    b1 = vv_last * 16 + m - 1

    def total_body(v, accv):
        return accv + load_acc(v)

    total_neg = lax.reduce_sum(
        lax.fori_loop(0, HBINS // 16, total_body,
                      jnp.zeros((16,), jnp.int32)), axes=(0,))
    lo1 = b1.astype(jnp.uint32) << jnp.uint32(LOW_BITS)
    lo1v = jnp.full((16,), lo1, dtype=jnp.uint32)

    _memset_i32(negbuf, CAP // 16)
    _memset_i32(posbuf, CAP // 16)
    offs[0] = jnp.int32(0)
    offs[1] = jnp.int32(0)

    def collect_vec(b, e16, key, valid=None):
        t = tbufs[b][pl.ds(e16, 16)]
        ge = key >= lo1v
        negm = (t == 0) & ge
        posm = (t == 1) & ge
        if valid is not None:
            negm = negm & valid
            posm = posm & valid
        noff = offs[0]
        poff = offs[1]
        plsc.store_compressed(negbuf.at[pl.ds(noff, 16)], key, mask=negm)
        plsc.store_compressed(posbuf.at[pl.ds(poff, 16)], key, mask=posm)
        nadd = lax.reduce_sum(jnp.where(negm, 1, 0), axes=(0,))
        padd = lax.reduce_sum(jnp.where(posm, 1, 0), axes=(0,))
        offs[0] = jnp.minimum(noff + nadd, CAP - 16)
        offs[1] = jnp.minimum(poff + padd, CAP - 16)

    def per_chunk2(b, nblk, is_tail):
        def blk(i, carry):
            keys = []
            mx = None
            for u in range(UNROLL):
                p = pbufs[b][pl.ds(i * BLK + u * 16, 16)]
                key = _key16(p)
                keys.append(key)
                mx = key if mx is None else jnp.maximum(mx, key)
            anyhit = lax.reduce_max(mx, axes=(0,)) >= lo1

            @pl.when(anyhit)
            def _():
                for u in range(UNROLL):
                    collect_vec(b, i * BLK + u * 16, keys[u])

            return carry

        lax.fori_loop(0, nblk, blk, 0)
        if is_tail:
            for u in range(TAIL_VECS):
                e16 = TAIL_BLKS * BLK + u * 16
                collect_vec(b, e16, _key16(pbufs[b][pl.ds(e16, 16)]))
            e16 = TAIL_BLKS * BLK + TAIL_VECS * 16
            collect_vec(b, e16, _key16(pbufs[b][pl.ds(e16, 16)]),
                        valid=lanes < TAIL_REM)

    _chunk_loop(preds_hbm, target_hbm, base, pbufs, tbufs,
                (sem0, sem1), per_chunk2)

    lo1_i32 = lax.bitcast_convert_type(lo1v, jnp.int32)
    meta = jnp.where(lanes == 0, offs[0], 0)
    meta = meta + jnp.where(lanes == 1, offs[1], 0)
    meta = meta + jnp.where(lanes == 2, lo1_i32, 0)
    meta = meta + jnp.where(lanes == 3, total_neg, 0)
    stage[pl.ds(0, 16)] = meta
    pltpu.sync_copy(negbuf, negk_out.at[w])
    pltpu.sync_copy(posbuf, posk_out.at[w])
    pltpu.sync_copy(stage, cnt_out.at[w])


def _select_body(negk_ref, posk_ref, cnt_ref, out_ref):
    negk = negk_ref[...]
    posk = posk_ref[...]
    cnt = cnt_ref[...]
    lo1 = jnp.maximum(cnt[0, 2], cnt[1, 2]).astype(jnp.uint32)
    num_pos = N - (cnt[0, 3] + cnt[1, 3])

    def count_gt(buf, thresh):
        return jnp.sum((buf > thresh).astype(jnp.int32))

    def bis(_, carry):
        lo, hi = carry
        mid = lo + ((hi - lo) >> jnp.uint32(1))
        c = count_gt(negk, mid)
        take_hi = c < K
        lo = jnp.where(take_hi, lo, mid + jnp.uint32(1))
        hi = jnp.where(take_hi, mid, hi)
        return lo, hi

    kth, _ = lax.fori_loop(0, 32, bis, (lo1, jnp.uint32(0xFFFFFFFF)))
    hits = count_gt(posk, kth)
    res = hits.astype(jnp.float32) / num_pos.astype(jnp.float32)
    out_ref[...] = jnp.full((1, 1), res, dtype=jnp.float32)


_select = pl.pallas_call(
    _select_body,
    out_shape=jax.ShapeDtypeStruct((1, 1), jnp.float32),
)


def kernel(preds, target):
    negk, posk, cnts = _scan(preds, target)
    out = _select(negk, posk, cnts)
    return out[0, 0]


# speedup vs baseline: 54.5568x; 1.0404x over previous
import functools

import jax
import jax.numpy as jnp
from jax import lax
from jax.experimental import pallas as pl
from jax.experimental.pallas import tpu as pltpu
from jax.experimental.pallas import tpu_sc as plsc

N = 4_000_000
K = 100
NC = 2
NS = 16
NW = NC * NS
PER_W = N // NW
FULL_CH = 16384
N_FULL = PER_W // FULL_CH
TAIL_CH = PER_W - N_FULL * FULL_CH
UNROLL = 8
BLK = UNROLL * 16
TAIL_BLKS = TAIL_CH // BLK
TAIL_VECS = (TAIL_CH - TAIL_BLKS * BLK) // 16
TAIL_REM = TAIL_CH - TAIL_BLKS * BLK - TAIL_VECS * 16
HBITS = 14
HBINS = 1 << HBITS
LOW_BITS = 32 - HBITS
SLICE = HBINS // NS
CAP = 256
GRP_VECS = 32
GRP = GRP_VECS * 16

_mesh = plsc.VectorSubcoreMesh(core_axis_name="c", subcore_axis_name="s")
_params = pltpu.CompilerParams(needs_layout_passes=False)


def _wid():
    return lax.axis_index("s") * NC + lax.axis_index("c")


def _key16(p):
    b = lax.bitcast_convert_type(p, jnp.uint32)
    top = b >> jnp.uint32(31)
    flip = (jnp.uint32(0) - top) | jnp.uint32(0x80000000)
    return b ^ flip


def _memset_i32(ref, nvecs, value=0):
    zz = jnp.full((16,), value, dtype=jnp.int32)

    def body(i, carry):
        ref[pl.ds(i * 16, 16)] = zz
        return carry

    lax.fori_loop(0, nvecs, body, 0)


def _chunk_loop(preds_hbm, target_hbm, base, pbufs, tbufs, sems, per_chunk):
    def issue(c):
        ch = FULL_CH if c < N_FULL else TAIL_CH
        off = base + c * FULL_CH
        b = c % 2
        dp = pltpu.async_copy(
            preds_hbm.at[pl.ds(off, ch)], pbufs[b].at[pl.ds(0, ch)], sems[b])
        dt = pltpu.async_copy(
            target_hbm.at[pl.ds(off, ch)], tbufs[b].at[pl.ds(0, ch)], sems[b])
        return dp, dt

    descs = [None, None]
    descs[0] = issue(0)
    for c in range(N_FULL + 1):
        if c + 1 <= N_FULL:
            descs[(c + 1) % 2] = issue(c + 1)
        dp, dt = descs[c % 2]
        dp.wait()
        dt.wait()
        nblk = (FULL_CH // BLK) if c < N_FULL else TAIL_BLKS
        per_chunk(c % 2, nblk, is_tail=(c == N_FULL))


@functools.partial(
    pl.kernel,
    out_type=(
        jax.ShapeDtypeStruct((NW, CAP), jnp.uint32),
        jax.ShapeDtypeStruct((NW, CAP), jnp.uint32),
        jax.ShapeDtypeStruct((NW, 16), jnp.int32),
    ),
    mesh=_mesh,
    compiler_params=_params,
    scratch_types=[
        pltpu.VMEM((FULL_CH,), jnp.float32),
        pltpu.VMEM((FULL_CH,), jnp.float32),
        pltpu.VMEM((FULL_CH,), jnp.int32),
        pltpu.VMEM((FULL_CH,), jnp.int32),
        pltpu.VMEM((HBINS,), jnp.int32),
        pltpu.VMEM((NS, SLICE), jnp.int32),
        pltpu.VMEM((CAP,), jnp.uint32),
        pltpu.VMEM((CAP,), jnp.uint32),
        pltpu.VMEM((16,), jnp.int32),
        pltpu.SMEM((8,), jnp.int32),
        pltpu.VMEM_SHARED((NS, HBINS), jnp.int32),
        pltpu.VMEM_SHARED((HBINS,), jnp.int32),
        pltpu.SemaphoreType.DMA,
        pltpu.SemaphoreType.DMA,
    ],
)
def _scan(preds_hbm, target_hbm, negk_out, posk_out, cnt_out,
          pbuf0, pbuf1, tbuf0, tbuf1, hist, colbuf, negbuf, posbuf, stage,
          offs, slab, merged_sh, sem0, sem1):
    w = _wid()
    sid = lax.axis_index("s")
    base = w * PER_W
    pbufs, tbufs = (pbuf0, pbuf1), (tbuf0, tbuf1)
    lanes = lax.iota(jnp.int32, 16)

    _memset_i32(hist, HBINS // 16)

    def calc_vec(b, e16, valid=None):
        p = pbufs[b][pl.ds(e16, 16)]
        t = tbufs[b][pl.ds(e16, 16)]
        key = _key16(p)
        bucket = (key >> jnp.uint32(LOW_BITS)).astype(jnp.int32)
        negm = t == 0
        if valid is not None:
            negm = negm & valid
        cnt, last = plsc.scan_count(bucket, mask=negm)
        return bucket, cnt, last

    def do_vec(b, e16, valid=None):
        bucket, cnt, last = calc_vec(b, e16, valid)
        plsc.addupdate_scatter(hist, [bucket], cnt, mask=last)

    def per_chunk1(b, nblk, is_tail):
        def blk(i, carry):
            pend = [calc_vec(b, i * BLK + u * 16) for u in range(UNROLL)]
            for bucket, cnt, last in pend:
                plsc.addupdate_scatter(hist, [bucket], cnt, mask=last)
            return carry

        lax.fori_loop(0, nblk, blk, 0)
        if is_tail:
            for u in range(TAIL_VECS):
                do_vec(b, TAIL_BLKS * BLK + u * 16)
            do_vec(b, TAIL_BLKS * BLK + TAIL_VECS * 16,
                   valid=lanes < TAIL_REM)

    _chunk_loop(preds_hbm, target_hbm, base, pbufs, tbufs,
                (sem0, sem1), per_chunk1)

    pltpu.sync_copy(hist, slab.at[sid])
    plsc.subcore_barrier()
    for r in range(NS):
        pltpu.sync_copy(slab.at[r, pl.ds(sid * SLICE, SLICE)], colbuf.at[r])

    def red(v, carry):
        acc = jnp.zeros((16,), dtype=jnp.int32)
        for r in range(NS):
            acc = acc + colbuf[r, pl.ds(v * 16, 16)]
        hist[pl.ds(sid * SLICE + v * 16, 16)] = acc
        return carry

    lax.fori_loop(0, SLICE // 16, red, 0)
    pltpu.sync_copy(hist.at[pl.ds(sid * SLICE, SLICE)],
                    merged_sh.at[pl.ds(sid * SLICE, SLICE)])
    plsc.subcore_barrier()
    pltpu.sync_copy(merged_sh, hist)

    def load_acc(vv):
        return hist[pl.ds(vv * 16, 16)]

    def walk_cond(carry):
        v, cnt_above = carry
        return (cnt_above < K) & (v < HBINS // 16)

    def walk_body(carry):
        v, cnt_above = carry
        acc = load_acc(HBINS // 16 - 1 - v)
        return v + 1, cnt_above + lax.reduce_sum(acc, axes=(0,))

    nv, cnt_incl = lax.while_loop(walk_cond, walk_body,
                                  (jnp.int32(0), jnp.int32(0)))
    vv_last = HBINS // 16 - nv
    acc = load_acc(vv_last)
    tot_last = lax.reduce_sum(acc, axes=(0,))
    suf = lax.rev(lax.cumsum(lax.rev(acc, (0,)), axis=0), (0,))
    cnt_ge = suf + jnp.full((16,), cnt_incl - tot_last, dtype=jnp.int32)
    m = lax.reduce_sum(jnp.where(cnt_ge >= K, 1, 0), axes=(0,))
    b1 = vv_last * 16 + m - 1

    def total_body(v, accv):
        return accv + load_acc(v)

    total_neg = lax.reduce_sum(
        lax.fori_loop(0, HBINS // 16, total_body,
                      jnp.zeros((16,), jnp.int32)), axes=(0,))
    lo1 = b1.astype(jnp.uint32) << jnp.uint32(LOW_BITS)
    lo1v = jnp.full((16,), lo1, dtype=jnp.uint32)

    _memset_i32(negbuf, CAP // 16)
    _memset_i32(posbuf, CAP // 16)
    offs[0] = jnp.int32(0)
    offs[1] = jnp.int32(0)

    def collect_vec(b, e16, key, valid=None):
        t = tbufs[b][pl.ds(e16, 16)]
        ge = key >= lo1v
        negm = (t == 0) & ge
        posm = (t == 1) & ge
        if valid is not None:
            negm = negm & valid
            posm = posm & valid
        noff = offs[0]
        poff = offs[1]
        plsc.store_compressed(negbuf.at[pl.ds(noff, 16)], key, mask=negm)
        plsc.store_compressed(posbuf.at[pl.ds(poff, 16)], key, mask=posm)
        nadd = lax.reduce_sum(jnp.where(negm, 1, 0), axes=(0,))
        padd = lax.reduce_sum(jnp.where(posm, 1, 0), axes=(0,))
        offs[0] = jnp.minimum(noff + nadd, CAP - 16)
        offs[1] = jnp.minimum(poff + padd, CAP - 16)

    def per_chunk2(b, nblk, is_tail):
        ngrp = nblk // (GRP_VECS // UNROLL)

        def grp(i, carry):
            mx = None
            for u in range(GRP_VECS):
                p = pbufs[b][pl.ds(i * GRP + u * 16, 16)]
                key = _key16(p)
                mx = key if mx is None else jnp.maximum(mx, key)
            anyhit = lax.reduce_max(mx, axes=(0,)) >= lo1

            @pl.when(anyhit)
            def _():
                for u in range(GRP_VECS):
                    e16 = i * GRP + u * 16
                    collect_vec(b, e16, _key16(pbufs[b][pl.ds(e16, 16)]))

            return carry

        lax.fori_loop(0, ngrp, grp, 0)
        if is_tail:
            for u in range(TAIL_VECS):
                e16 = TAIL_BLKS * BLK + u * 16
                collect_vec(b, e16, _key16(pbufs[b][pl.ds(e16, 16)]))
            e16 = TAIL_BLKS * BLK + TAIL_VECS * 16
            collect_vec(b, e16, _key16(pbufs[b][pl.ds(e16, 16)]),
                        valid=lanes < TAIL_REM)

    _chunk_loop(preds_hbm, target_hbm, base, pbufs, tbufs,
                (sem0, sem1), per_chunk2)

    lo1_i32 = lax.bitcast_convert_type(lo1v, jnp.int32)
    meta = jnp.where(lanes == 0, offs[0], 0)
    meta = meta + jnp.where(lanes == 1, offs[1], 0)
    meta = meta + jnp.where(lanes == 2, lo1_i32, 0)
    meta = meta + jnp.where(lanes == 3, total_neg, 0)
    stage[pl.ds(0, 16)] = meta
    pltpu.sync_copy(negbuf, negk_out.at[w])
    pltpu.sync_copy(posbuf, posk_out.at[w])
    pltpu.sync_copy(stage, cnt_out.at[w])


def _select_body(negk_ref, posk_ref, cnt_ref, out_ref):
    negk = negk_ref[...]
    posk = posk_ref[...]
    cnt = cnt_ref[...]
    lo1 = jnp.maximum(cnt[0, 2], cnt[1, 2]).astype(jnp.uint32)
    num_pos = N - (cnt[0, 3] + cnt[1, 3])

    def count_gt(buf, thresh):
        return jnp.sum((buf > thresh).astype(jnp.int32))

    def bis(_, carry):
        lo, hi = carry
        mid = lo + ((hi - lo) >> jnp.uint32(1))
        c = count_gt(negk, mid)
        take_hi = c < K
        lo = jnp.where(take_hi, lo, mid + jnp.uint32(1))
        hi = jnp.where(take_hi, mid, hi)
        return lo, hi

    kth, _ = lax.fori_loop(0, 32, bis, (lo1, jnp.uint32(0xFFFFFFFF)))
    hits = count_gt(posk, kth)
    res = hits.astype(jnp.float32) / num_pos.astype(jnp.float32)
    out_ref[...] = jnp.full((1, 1), res, dtype=jnp.float32)


_select = pl.pallas_call(
    _select_body,
    out_shape=jax.ShapeDtypeStruct((1, 1), jnp.float32),
)


def kernel(preds, target):
    negk, posk, cnts = _scan(preds, target)
    out = _select(negk, posk, cnts)
    return out[0, 0]
